# baseline (speedup 1.0000x reference)
import jax
import jax.numpy as jnp
from jax import lax
from jax.experimental import pallas as pl
from jax.experimental.pallas import tpu as pltpu
from jax.experimental.pallas import tpu_sc as plsc

N = 50000
D = 256
G = 128
LANES = 16
CG = D // LANES
K = 64
RB = 128
NEG_INF = float("-inf")

_info = plsc.get_sparse_core_info()
NC, NS = _info.num_cores, _info.num_subcores
NW = NC * NS
G_SC = NW
G_TC = G - G_SC
STARTS_PAD = G + LANES


def _sc_body(x_hbm, starts_hbm, out_hbm, starts_v, buf0, buf1,
             out_v, sem0, sem1):
    wid = lax.axis_index("s") * NC + lax.axis_index("c")

    pltpu.sync_copy(starts_hbm, starts_v)
    win = starts_v[pl.ds(wid, LANES)]
    s = win[0]
    e = win[1]
    s_al = (s // 8) * 8
    nch = (e - s_al + (K - 1)) // K
    npair = (nch + 1) // 2

    def chunk_base(ci):
        return pl.multiple_of(jnp.minimum(s_al + ci * K, N - K), 8)

    def start_copy(ci, buf, sem):
        src = x_hbm.at[pl.ds(chunk_base(ci), K), :]
        pltpu.make_async_copy(src, buf, sem).start()

    def wait_copy(ci, buf, sem):
        src = x_hbm.at[pl.ds(chunk_base(ci), K), :]
        pltpu.make_async_copy(src, buf, sem).wait()

    def reduce_chunk(accs, ci, buf):
        base = chunk_base(ci)
        j_lo = jnp.maximum(s - base, 0)
        j_hi = jnp.clip(e - base, 0, K)
        j_hi = jnp.where(ci < nch, j_hi, 0)

        def row_body(j, accs):
            return tuple(
                jnp.maximum(accs[c], buf[j, c * LANES:(c + 1) * LANES])
                for c in range(CG)
            )

        return lax.fori_loop(j_lo, j_hi, row_body, accs)

    @pl.when(nch > 0)
    def _():
        start_copy(0, buf0, sem0)

    def pair_body(p, accs):
        c0 = 2 * p
        @pl.when(c0 + 1 < nch)
        def _():
            start_copy(c0 + 1, buf1, sem1)
        wait_copy(c0, buf0, sem0)
        accs = reduce_chunk(accs, c0, buf0)
        @pl.when(c0 + 2 < nch)
        def _():
            start_copy(c0 + 2, buf0, sem0)
        @pl.when(c0 + 1 < nch)
        def _():
            wait_copy(c0 + 1, buf1, sem1)
        accs = reduce_chunk(accs, c0 + 1, buf1)
        return accs

    acc0 = tuple(jnp.full((LANES,), NEG_INF, jnp.float32) for _ in range(CG))
    accs = lax.fori_loop(0, npair, pair_body, acc0)
    for c in range(CG):
        out_v[0, c * LANES:(c + 1) * LANES] = accs[c]

    pltpu.sync_copy(out_v, out_hbm.at[wid])


NBLK = (N + RB - 1) // RB
MAXC = G_TC + NBLK


def _tc_body(seg_r, blk_r, cs_r, ce_r, first_r, x_blk, out_v):
    ci = pl.program_id(0)
    s = cs_r[ci]
    e = ce_r[ci]
    base = blk_r[ci] * RB
    rows = base + lax.broadcasted_iota(jnp.int32, (RB, 1), 0)
    valid = jnp.logical_and(rows >= s, rows < e)
    xm = jnp.where(valid, x_blk[...], NEG_INF)
    partial = jnp.max(xm.reshape(RB // 8, 8, D), axis=0)
    partial = jnp.max(partial, axis=0, keepdims=True).reshape(1, 1, D)
    first = first_r[ci] != 0

    @pl.when(first)
    def _():
        out_v[...] = partial

    @pl.when(jnp.logical_not(first))
    def _():
        out_v[...] = jnp.maximum(out_v[...], partial)


@jax.jit
def kernel(x, batch):
    starts = jnp.searchsorted(
        batch, jnp.arange(G + 1, dtype=jnp.int32), method="compare_all"
    ).astype(jnp.int32)
    starts = jnp.concatenate(
        [starts, jnp.full((STARTS_PAD - (G + 1),), N, jnp.int32)])

    sc_fn = pl.kernel(
        _sc_body,
        out_type=jax.ShapeDtypeStruct((NW, 1, D), jnp.float32),
        mesh=plsc.VectorSubcoreMesh(core_axis_name="c", subcore_axis_name="s"),
        scratch_types=[
            pltpu.VMEM((STARTS_PAD,), jnp.int32),
            pltpu.VMEM((K, D), jnp.float32),
            pltpu.VMEM((K, D), jnp.float32),
            pltpu.VMEM((1, D), jnp.float32),
            pltpu.SemaphoreType.DMA,
            pltpu.SemaphoreType.DMA,
        ],
    )
    sc_out = sc_fn(x, starts)

    s_g = starts[G_SC:G]
    e_g = starts[G_SC + 1:G + 1]
    b0 = s_g // RB
    nb = jnp.maximum((e_g - b0 * RB + (RB - 1)) // RB, 1)
    off = jnp.concatenate([jnp.zeros((1,), jnp.int32), jnp.cumsum(nb)])
    total = off[G_TC]
    ci = jnp.arange(MAXC, dtype=jnp.int32)
    seg_of = jnp.clip(
        jnp.searchsorted(off, ci, side="right").astype(jnp.int32) - 1,
        0, G_TC - 1)
    within = ci - off[seg_of]
    blk = jnp.clip(b0[seg_of] + within, 0, NBLK - 1).astype(jnp.int32)
    is_real = ci < total
    cs = jnp.where(is_real, s_g[seg_of], 0).astype(jnp.int32)
    ce = jnp.where(is_real, e_g[seg_of], 0).astype(jnp.int32)
    first = jnp.logical_and(within == 0, is_real).astype(jnp.int32)

    tc_out = pl.pallas_call(
        _tc_body,
        out_shape=jax.ShapeDtypeStruct((G_TC, 1, D), jnp.float32),
        grid_spec=pltpu.PrefetchScalarGridSpec(
            num_scalar_prefetch=5,
            grid=(MAXC,),
            in_specs=[
                pl.BlockSpec((RB, D),
                             lambda i, seg_r, blk_r, cs_r, ce_r, first_r:
                             (blk_r[i], 0)),
            ],
            out_specs=pl.BlockSpec((1, 1, D),
                                   lambda i, seg_r, blk_r, cs_r, ce_r, first_r:
                                   (seg_r[i], 0, 0)),
        ),
    )(seg_of, blk, cs, ce, first, x)

    return jnp.concatenate([sc_out.reshape(G_SC, D),
                            tc_out.reshape(G_TC, D)], axis=0)


# speedup vs baseline: 1.0028x; 1.0028x over previous
import jax
import jax.numpy as jnp
from jax import lax
from jax.experimental import pallas as pl
from jax.experimental.pallas import tpu as pltpu
from jax.experimental.pallas import tpu_sc as plsc

N = 50000
D = 256
G = 128
LANES = 16
CG = D // LANES
K = 64
RB = 128
NEG_INF = float("-inf")

_info = plsc.get_sparse_core_info()
NC, NS = _info.num_cores, _info.num_subcores
NW = NC * NS
G_SC = NW
G_TC = G - G_SC
STARTS_PAD = G + LANES


def _sc_body(x_hbm, starts_hbm, out_hbm, starts_v, buf0, buf1,
             out_v, sem0, sem1):
    wid = lax.axis_index("s") * NC + lax.axis_index("c")

    pltpu.sync_copy(starts_hbm, starts_v)
    win = starts_v[pl.ds(wid, LANES)]
    s = win[0]
    e = win[1]
    s_al = (s // 8) * 8
    nch = (e - s_al + (K - 1)) // K
    npair = (nch + 1) // 2

    def chunk_base(ci):
        return pl.multiple_of(jnp.minimum(s_al + ci * K, N - K), 8)

    def start_copy(ci, buf, sem):
        src = x_hbm.at[pl.ds(chunk_base(ci), K), :]
        pltpu.make_async_copy(src, buf, sem).start()

    def wait_copy(ci, buf, sem):
        src = x_hbm.at[pl.ds(chunk_base(ci), K), :]
        pltpu.make_async_copy(src, buf, sem).wait()

    def reduce_chunk(accs, ci, buf):
        base = chunk_base(ci)
        j_lo = jnp.maximum(s - base, 0)
        j_hi = jnp.clip(e - base, 0, K)
        j_hi = jnp.where(ci < nch, j_hi, 0)

        def row_body(j, accs):
            return tuple(
                jnp.maximum(accs[c], buf[j, c * LANES:(c + 1) * LANES])
                for c in range(CG)
            )

        return lax.fori_loop(j_lo, j_hi, row_body, accs)

    @pl.when(nch > 0)
    def _():
        start_copy(0, buf0, sem0)

    def pair_body(p, accs):
        c0 = 2 * p
        @pl.when(c0 + 1 < nch)
        def _():
            start_copy(c0 + 1, buf1, sem1)
        wait_copy(c0, buf0, sem0)
        accs = reduce_chunk(accs, c0, buf0)
        @pl.when(c0 + 2 < nch)
        def _():
            start_copy(c0 + 2, buf0, sem0)
        @pl.when(c0 + 1 < nch)
        def _():
            wait_copy(c0 + 1, buf1, sem1)
        accs = reduce_chunk(accs, c0 + 1, buf1)
        return accs

    acc0 = tuple(jnp.full((LANES,), NEG_INF, jnp.float32) for _ in range(CG))
    accs = lax.fori_loop(0, npair, pair_body, acc0)
    for c in range(CG):
        out_v[0, c * LANES:(c + 1) * LANES] = accs[c]

    pltpu.sync_copy(out_v, out_hbm.at[wid])


NBLK = (N + RB - 1) // RB
MAXC = G_TC + NBLK


def _tc_body(seg_r, blk_r, cs_r, ce_r, first_r, x_blk, out_v):
    ci = pl.program_id(0)
    s = cs_r[ci]
    e = ce_r[ci]
    base = blk_r[ci] * RB
    rows = base + lax.broadcasted_iota(jnp.int32, (RB, 1), 0)
    valid = jnp.logical_and(rows >= s, rows < e)
    xm = jnp.where(valid, x_blk[...], NEG_INF)
    m = xm[0:8, :]
    for k in range(1, RB // 8):
        m = jnp.maximum(m, xm[8 * k:8 * k + 8, :])
    partial = jnp.max(m, axis=0, keepdims=True).reshape(1, 1, D)
    first = first_r[ci] != 0

    @pl.when(first)
    def _():
        out_v[...] = partial

    @pl.when(jnp.logical_not(first))
    def _():
        out_v[...] = jnp.maximum(out_v[...], partial)


@jax.jit
def kernel(x, batch):
    starts = jnp.searchsorted(
        batch, jnp.arange(G + 1, dtype=jnp.int32), method="compare_all"
    ).astype(jnp.int32)
    starts = jnp.concatenate(
        [starts, jnp.full((STARTS_PAD - (G + 1),), N, jnp.int32)])

    sc_fn = pl.kernel(
        _sc_body,
        out_type=jax.ShapeDtypeStruct((NW, 1, D), jnp.float32),
        mesh=plsc.VectorSubcoreMesh(core_axis_name="c", subcore_axis_name="s"),
        scratch_types=[
            pltpu.VMEM((STARTS_PAD,), jnp.int32),
            pltpu.VMEM((K, D), jnp.float32),
            pltpu.VMEM((K, D), jnp.float32),
            pltpu.VMEM((1, D), jnp.float32),
            pltpu.SemaphoreType.DMA,
            pltpu.SemaphoreType.DMA,
        ],
    )
    sc_out = sc_fn(x, starts)

    s_g = starts[G_SC:G]
    e_g = starts[G_SC + 1:G + 1]
    b0 = s_g // RB
    nb = jnp.maximum((e_g - b0 * RB + (RB - 1)) // RB, 1)
    off = jnp.concatenate([jnp.zeros((1,), jnp.int32), jnp.cumsum(nb)])
    total = off[G_TC]
    ci = jnp.arange(MAXC, dtype=jnp.int32)
    seg_of = jnp.clip(
        jnp.searchsorted(off, ci, side="right").astype(jnp.int32) - 1,
        0, G_TC - 1)
    within = ci - off[seg_of]
    blk = jnp.clip(b0[seg_of] + within, 0, NBLK - 1).astype(jnp.int32)
    is_real = ci < total
    cs = jnp.where(is_real, s_g[seg_of], 0).astype(jnp.int32)
    ce = jnp.where(is_real, e_g[seg_of], 0).astype(jnp.int32)
    first = jnp.logical_and(within == 0, is_real).astype(jnp.int32)

    tc_out = pl.pallas_call(
        _tc_body,
        out_shape=jax.ShapeDtypeStruct((G_TC, 1, D), jnp.float32),
        grid_spec=pltpu.PrefetchScalarGridSpec(
            num_scalar_prefetch=5,
            grid=(MAXC,),
            in_specs=[
                pl.BlockSpec((RB, D),
                             lambda i, seg_r, blk_r, cs_r, ce_r, first_r:
                             (blk_r[i], 0)),
            ],
            out_specs=pl.BlockSpec((1, 1, D),
                                   lambda i, seg_r, blk_r, cs_r, ce_r, first_r:
                                   (seg_r[i], 0, 0)),
        ),
    )(seg_of, blk, cs, ce, first, x)

    return jnp.concatenate([sc_out.reshape(G_SC, D),
                            tc_out.reshape(G_TC, D)], axis=0)


# speedup vs baseline: 2.2741x; 2.2676x over previous
import jax
import jax.numpy as jnp
from jax import lax
from jax.experimental import pallas as pl
from jax.experimental.pallas import tpu as pltpu
from jax.experimental.pallas import tpu_sc as plsc

N = 50000
D = 256
G = 128
LANES = 16
CG = D // LANES
K = 64
NEG_INF = float("-inf")

_info = plsc.get_sparse_core_info()
NC, NS = _info.num_cores, _info.num_subcores
NW = NC * NS
G_SC = NW
G_TC = G - G_SC
STARTS_PAD = G + LANES

RB = 128
NBUF = 16
NBLK = (N + RB - 1) // RB
MAXC = -((G_TC + NBLK) // -NBUF) * NBUF


def _sc_body(x_hbm, starts_hbm, out_hbm, starts_v, buf0, buf1,
             out_v, sem0, sem1):
    wid = lax.axis_index("s") * NC + lax.axis_index("c")

    pltpu.sync_copy(starts_hbm, starts_v)
    win = starts_v[pl.ds(wid, LANES)]
    s = win[0]
    e = win[1]
    s_al = (s // 8) * 8
    nch = (e - s_al + (K - 1)) // K
    npair = (nch + 1) // 2

    def chunk_base(ci):
        return pl.multiple_of(jnp.minimum(s_al + ci * K, N - K), 8)

    def start_copy(ci, buf, sem):
        src = x_hbm.at[pl.ds(chunk_base(ci), K), :]
        pltpu.make_async_copy(src, buf, sem).start()

    def wait_copy(ci, buf, sem):
        src = x_hbm.at[pl.ds(chunk_base(ci), K), :]
        pltpu.make_async_copy(src, buf, sem).wait()

    def reduce_chunk(accs, ci, buf):
        base = chunk_base(ci)
        j_lo = jnp.maximum(s - base, 0)
        j_hi = jnp.clip(e - base, 0, K)
        j_hi = jnp.where(ci < nch, j_hi, 0)

        def row_body(j, accs):
            return tuple(
                jnp.maximum(accs[c], buf[j, c * LANES:(c + 1) * LANES])
                for c in range(CG)
            )

        return lax.fori_loop(j_lo, j_hi, row_body, accs)

    @pl.when(nch > 0)
    def _():
        start_copy(0, buf0, sem0)

    def pair_body(p, accs):
        c0 = 2 * p
        @pl.when(c0 + 1 < nch)
        def _():
            start_copy(c0 + 1, buf1, sem1)
        wait_copy(c0, buf0, sem0)
        accs = reduce_chunk(accs, c0, buf0)
        @pl.when(c0 + 2 < nch)
        def _():
            start_copy(c0 + 2, buf0, sem0)
        @pl.when(c0 + 1 < nch)
        def _():
            wait_copy(c0 + 1, buf1, sem1)
        accs = reduce_chunk(accs, c0 + 1, buf1)
        return accs

    acc0 = tuple(jnp.full((LANES,), NEG_INF, jnp.float32) for _ in range(CG))
    accs = lax.fori_loop(0, npair, pair_body, acc0)
    for c in range(CG):
        out_v[0, c * LANES:(c + 1) * LANES] = accs[c]

    pltpu.sync_copy(out_v, out_hbm.at[wid])


def _tc_body(base_r, cs_r, ce_r, seg_r, flag_r, x_any, out_v, bufs, sems):
    def start_copy(ci, u):
        base = pl.multiple_of(base_r[ci], 8)
        src = x_any.at[pl.ds(base, RB), :]
        pltpu.make_async_copy(src, bufs.at[u], sems.at[u]).start()

    def wait_copy(ci, u):
        base = pl.multiple_of(base_r[ci], 8)
        src = x_any.at[pl.ds(base, RB), :]
        pltpu.make_async_copy(src, bufs.at[u], sems.at[u]).wait()

    for u in range(NBUF):
        start_copy(u, u)

    def reduce_chunk(ci, u, acc):
        base = base_r[ci]
        j_lo = jnp.clip(cs_r[ci] - base, 0, RB)
        j_hi = jnp.clip(ce_r[ci] - base, 0, RB)
        sub = lax.broadcasted_iota(jnp.int32, (8, 1), 0)
        part = jnp.full((8, D), NEG_INF, jnp.float32)
        for k in range(RB // 8):
            rows = 8 * k + sub
            valid = jnp.logical_and(rows >= j_lo, rows < j_hi)
            part = jnp.maximum(part,
                               jnp.where(valid, bufs[u, 8 * k:8 * k + 8, :],
                                         NEG_INF))
        flag = flag_r[ci]
        acc = jnp.where((flag & 1) != 0, part, jnp.maximum(acc, part))

        @pl.when((flag & 2) != 0)
        def _():
            out_v[pl.ds(seg_r[ci], 1), :, :] = acc.reshape(1, 8, D)

        return acc

    def group_body(p, acc):
        c0 = p * NBUF
        for u in range(NBUF):
            ci = c0 + u
            wait_copy(ci, u)
            acc = reduce_chunk(ci, u, acc)
            @pl.when(ci + NBUF < MAXC)
            def _():
                start_copy(ci + NBUF, u)
        return acc

    acc0 = jnp.full((8, D), NEG_INF, jnp.float32)
    lax.fori_loop(0, MAXC // NBUF, group_body, acc0)


@jax.jit
def kernel(x, batch):
    starts = jnp.searchsorted(
        batch, jnp.arange(G + 1, dtype=jnp.int32), method="compare_all"
    ).astype(jnp.int32)
    starts_pad = jnp.concatenate(
        [starts, jnp.full((STARTS_PAD - (G + 1),), N, jnp.int32)])

    sc_fn = pl.kernel(
        _sc_body,
        out_type=jax.ShapeDtypeStruct((NW, 1, D), jnp.float32),
        mesh=plsc.VectorSubcoreMesh(core_axis_name="c", subcore_axis_name="s"),
        scratch_types=[
            pltpu.VMEM((STARTS_PAD,), jnp.int32),
            pltpu.VMEM((K, D), jnp.float32),
            pltpu.VMEM((K, D), jnp.float32),
            pltpu.VMEM((1, D), jnp.float32),
            pltpu.SemaphoreType.DMA,
            pltpu.SemaphoreType.DMA,
        ],
    )
    sc_out = sc_fn(x, starts_pad)

    s_g = starts[G_SC:G]
    e_g = starts[G_SC + 1:G + 1]
    b0 = (s_g // 8) * 8
    nb = jnp.maximum((e_g - b0 + (RB - 1)) // RB, 1)
    off = jnp.concatenate([jnp.zeros((1,), jnp.int32), jnp.cumsum(nb)])
    total = off[G_TC]
    ci = jnp.arange(MAXC, dtype=jnp.int32)
    seg_of = jnp.clip(
        jnp.searchsorted(off, ci, side="right").astype(jnp.int32) - 1,
        0, G_TC - 1)
    within = ci - off[seg_of]
    base = jnp.clip(b0[seg_of] + within * RB, 0, ((N - RB) // 8) * 8)
    base = base.astype(jnp.int32)
    is_real = ci < total
    cs = jnp.where(is_real, s_g[seg_of], 0).astype(jnp.int32)
    ce = jnp.where(is_real, e_g[seg_of], 0).astype(jnp.int32)
    first = jnp.logical_and(within == 0, is_real)
    last = jnp.logical_and(within == nb[seg_of] - 1, is_real)
    flags = (first.astype(jnp.int32) + 2 * last.astype(jnp.int32))

    tc_out = pl.pallas_call(
        _tc_body,
        out_shape=jax.ShapeDtypeStruct((G_TC, 8, D), jnp.float32),
        in_specs=[
            pl.BlockSpec(memory_space=pltpu.SMEM),
            pl.BlockSpec(memory_space=pltpu.SMEM),
            pl.BlockSpec(memory_space=pltpu.SMEM),
            pl.BlockSpec(memory_space=pltpu.SMEM),
            pl.BlockSpec(memory_space=pltpu.SMEM),
            pl.BlockSpec(memory_space=pl.ANY),
        ],
        out_specs=pl.BlockSpec((G_TC, 8, D), lambda: (0, 0, 0)),
        scratch_shapes=[
            pltpu.VMEM((NBUF, RB, D), jnp.float32),
            pltpu.SemaphoreType.DMA((NBUF,)),
        ],
    )(base, cs, ce, seg_of, flags, x)

    return jnp.concatenate([sc_out.reshape(G_SC, D),
                            jnp.max(tc_out, axis=1)], axis=0)


# speedup vs baseline: 2.4369x; 1.0716x over previous
import jax
import jax.numpy as jnp
from jax import lax
from jax.experimental import pallas as pl
from jax.experimental.pallas import tpu as pltpu
from jax.experimental.pallas import tpu_sc as plsc

N = 50000
D = 256
G = 128
LANES = 16
CG = D // LANES
K = 64
NEG_INF = float("-inf")

_info = plsc.get_sparse_core_info()
NC, NS = _info.num_cores, _info.num_subcores
NW = NC * NS
G_SC = NW
G_TC = G - G_SC
STARTS_PAD = G + LANES

RB = 256
NBUF = 8
NBLK = (N + RB - 1) // RB
MAXC = -((G_TC + NBLK) // -NBUF) * NBUF


def _sc_body(x_hbm, starts_hbm, out_hbm, starts_v, buf0, buf1,
             out_v, sem0, sem1):
    wid = lax.axis_index("s") * NC + lax.axis_index("c")

    pltpu.sync_copy(starts_hbm, starts_v)
    win = starts_v[pl.ds(wid, LANES)]
    s = win[0]
    e = win[1]
    s_al = (s // 8) * 8
    nch = (e - s_al + (K - 1)) // K
    npair = (nch + 1) // 2

    def chunk_base(ci):
        return pl.multiple_of(jnp.minimum(s_al + ci * K, N - K), 8)

    def start_copy(ci, buf, sem):
        src = x_hbm.at[pl.ds(chunk_base(ci), K), :]
        pltpu.make_async_copy(src, buf, sem).start()

    def wait_copy(ci, buf, sem):
        src = x_hbm.at[pl.ds(chunk_base(ci), K), :]
        pltpu.make_async_copy(src, buf, sem).wait()

    def reduce_chunk(accs, ci, buf):
        base = chunk_base(ci)
        j_lo = jnp.maximum(s - base, 0)
        j_hi = jnp.clip(e - base, 0, K)
        j_hi = jnp.where(ci < nch, j_hi, 0)

        def row_body(j, accs):
            return tuple(
                jnp.maximum(accs[c], buf[j, c * LANES:(c + 1) * LANES])
                for c in range(CG)
            )

        return lax.fori_loop(j_lo, j_hi, row_body, accs)

    @pl.when(nch > 0)
    def _():
        start_copy(0, buf0, sem0)

    def pair_body(p, accs):
        c0 = 2 * p
        @pl.when(c0 + 1 < nch)
        def _():
            start_copy(c0 + 1, buf1, sem1)
        wait_copy(c0, buf0, sem0)
        accs = reduce_chunk(accs, c0, buf0)
        @pl.when(c0 + 2 < nch)
        def _():
            start_copy(c0 + 2, buf0, sem0)
        @pl.when(c0 + 1 < nch)
        def _():
            wait_copy(c0 + 1, buf1, sem1)
        accs = reduce_chunk(accs, c0 + 1, buf1)
        return accs

    acc0 = tuple(jnp.full((LANES,), NEG_INF, jnp.float32) for _ in range(CG))
    accs = lax.fori_loop(0, npair, pair_body, acc0)
    for c in range(CG):
        out_v[0, c * LANES:(c + 1) * LANES] = accs[c]

    pltpu.sync_copy(out_v, out_hbm.at[wid])


def _tc_body(base_r, cs_r, ce_r, seg_r, flag_r, x_any, out_v, bufs, sems):
    def start_copy(ci, u):
        base = pl.multiple_of(base_r[ci], 8)
        src = x_any.at[pl.ds(base, RB), :]
        pltpu.make_async_copy(src, bufs.at[u], sems.at[u]).start()

    def wait_copy(ci, u):
        base = pl.multiple_of(base_r[ci], 8)
        src = x_any.at[pl.ds(base, RB), :]
        pltpu.make_async_copy(src, bufs.at[u], sems.at[u]).wait()

    for u in range(NBUF):
        start_copy(u, u)

    def reduce_chunk(ci, u, acc):
        base = base_r[ci]
        j_lo = jnp.clip(cs_r[ci] - base, 0, RB)
        j_hi = jnp.clip(ce_r[ci] - base, 0, RB)
        sub = lax.broadcasted_iota(jnp.int32, (8, 1), 0)
        part = jnp.full((8, D), NEG_INF, jnp.float32)
        for k in range(RB // 8):
            rows = 8 * k + sub
            valid = jnp.logical_and(rows >= j_lo, rows < j_hi)
            part = jnp.maximum(part,
                               jnp.where(valid, bufs[u, 8 * k:8 * k + 8, :],
                                         NEG_INF))
        flag = flag_r[ci]
        acc = jnp.where((flag & 1) != 0, part, jnp.maximum(acc, part))

        @pl.when((flag & 2) != 0)
        def _():
            out_v[pl.ds(seg_r[ci], 1), :, :] = acc.reshape(1, 8, D)

        return acc

    def group_body(p, acc):
        c0 = p * NBUF
        for u in range(NBUF):
            ci = c0 + u
            wait_copy(ci, u)
            acc = reduce_chunk(ci, u, acc)
            @pl.when(ci + NBUF < MAXC)
            def _():
                start_copy(ci + NBUF, u)
        return acc

    acc0 = jnp.full((8, D), NEG_INF, jnp.float32)
    lax.fori_loop(0, MAXC // NBUF, group_body, acc0)


@jax.jit
def kernel(x, batch):
    starts = jnp.searchsorted(
        batch, jnp.arange(G + 1, dtype=jnp.int32), method="compare_all"
    ).astype(jnp.int32)
    starts_pad = jnp.concatenate(
        [starts, jnp.full((STARTS_PAD - (G + 1),), N, jnp.int32)])

    sc_fn = pl.kernel(
        _sc_body,
        out_type=jax.ShapeDtypeStruct((NW, 1, D), jnp.float32),
        mesh=plsc.VectorSubcoreMesh(core_axis_name="c", subcore_axis_name="s"),
        scratch_types=[
            pltpu.VMEM((STARTS_PAD,), jnp.int32),
            pltpu.VMEM((K, D), jnp.float32),
            pltpu.VMEM((K, D), jnp.float32),
            pltpu.VMEM((1, D), jnp.float32),
            pltpu.SemaphoreType.DMA,
            pltpu.SemaphoreType.DMA,
        ],
    )
    sc_out = sc_fn(x, starts_pad)

    s_g = starts[G_SC:G]
    e_g = starts[G_SC + 1:G + 1]
    b0 = (s_g // 8) * 8
    nb = jnp.maximum((e_g - b0 + (RB - 1)) // RB, 1)
    off = jnp.concatenate([jnp.zeros((1,), jnp.int32), jnp.cumsum(nb)])
    total = off[G_TC]
    ci = jnp.arange(MAXC, dtype=jnp.int32)
    seg_of = jnp.clip(
        jnp.searchsorted(off, ci, side="right").astype(jnp.int32) - 1,
        0, G_TC - 1)
    within = ci - off[seg_of]
    base = jnp.clip(b0[seg_of] + within * RB, 0, ((N - RB) // 8) * 8)
    base = base.astype(jnp.int32)
    is_real = ci < total
    cs = jnp.where(is_real, s_g[seg_of], 0).astype(jnp.int32)
    ce = jnp.where(is_real, e_g[seg_of], 0).astype(jnp.int32)
    first = jnp.logical_and(within == 0, is_real)
    last = jnp.logical_and(within == nb[seg_of] - 1, is_real)
    flags = (first.astype(jnp.int32) + 2 * last.astype(jnp.int32))

    tc_out = pl.pallas_call(
        _tc_body,
        out_shape=jax.ShapeDtypeStruct((G_TC, 8, D), jnp.float32),
        in_specs=[
            pl.BlockSpec(memory_space=pltpu.SMEM),
            pl.BlockSpec(memory_space=pltpu.SMEM),
            pl.BlockSpec(memory_space=pltpu.SMEM),
            pl.BlockSpec(memory_space=pltpu.SMEM),
            pl.BlockSpec(memory_space=pltpu.SMEM),
            pl.BlockSpec(memory_space=pl.ANY),
        ],
        out_specs=pl.BlockSpec((G_TC, 8, D), lambda: (0, 0, 0)),
        scratch_shapes=[
            pltpu.VMEM((NBUF, RB, D), jnp.float32),
            pltpu.SemaphoreType.DMA((NBUF,)),
        ],
    )(base, cs, ce, seg_of, flags, x)

    return jnp.concatenate([sc_out.reshape(G_SC, D),
                            jnp.max(tc_out, axis=1)], axis=0)


# speedup vs baseline: 4.9245x; 2.0208x over previous
import jax
import jax.numpy as jnp
from jax import lax
from jax.experimental import pallas as pl
from jax.experimental.pallas import tpu as pltpu
from jax.experimental.pallas import tpu_sc as plsc

N = 50000
D = 256
G = 128
LANES = 16
CG = D // LANES
K = 64
NEG_INF = float("-inf")

_info = plsc.get_sparse_core_info()
NC, NS = _info.num_cores, _info.num_subcores
NW = NC * NS
SEG_PER_W = G // NW
STARTS_PAD = G + LANES


def _seg_max_body(x_hbm, starts_hbm, out_hbm, starts_v, buf0, buf1, buf2,
                  out_v, sem0, sem1, sem2):
    wid = lax.axis_index("s") * NC + lax.axis_index("c")
    g0 = wid * SEG_PER_W

    pltpu.sync_copy(starts_hbm, starts_v)
    win = starts_v[pl.ds(g0, LANES)]

    def chunk_base(s_al, ci):
        return pl.multiple_of(jnp.minimum(s_al + ci * K, N - K), 8)

    def start_copy(s_al, ci, buf, sem):
        src = x_hbm.at[pl.ds(chunk_base(s_al, ci), K), :]
        pltpu.make_async_copy(src, buf, sem).start()

    def wait_copy(s_al, ci, buf, sem):
        src = x_hbm.at[pl.ds(chunk_base(s_al, ci), K), :]
        pltpu.make_async_copy(src, buf, sem).wait()

    def reduce_chunk(accs, s, e, s_al, nch, ci, buf):
        base = chunk_base(s_al, ci)
        j_lo = jnp.maximum(s - base, 0)
        j_hi = jnp.clip(e - base, 0, K)
        j_hi = jnp.where(ci < nch, j_hi, 0)

        def row_body(j, accs):
            return tuple(
                jnp.maximum(accs[c], buf[j, c * LANES:(c + 1) * LANES])
                for c in range(CG)
            )

        return lax.fori_loop(j_lo, j_hi, row_body, accs)

    for gl in range(SEG_PER_W):
        s = win[gl]
        e = win[gl + 1]
        s_al = (s // 8) * 8
        nch = (e - s_al + (K - 1)) // K
        ntri = (nch + 2) // 3

        for i, (b, sm) in enumerate(((buf0, sem0), (buf1, sem1),
                                     (buf2, sem2))):
            @pl.when(nch > i)
            def _(i=i, b=b, sm=sm):
                start_copy(s_al, i, b, sm)

        def tri_body(p, accs, s=s, e=e, s_al=s_al, nch=nch):
            c0 = 3 * p
            for i, (b, sm) in enumerate(((buf0, sem0), (buf1, sem1),
                                         (buf2, sem2))):
                ci = c0 + i
                @pl.when(ci < nch)
                def _(ci=ci, b=b, sm=sm):
                    wait_copy(s_al, ci, b, sm)
                accs = reduce_chunk(accs, s, e, s_al, nch, ci, b)
                @pl.when(ci + 3 < nch)
                def _(ci=ci, b=b, sm=sm):
                    start_copy(s_al, ci + 3, b, sm)
            return accs

        acc0 = tuple(jnp.full((LANES,), NEG_INF, jnp.float32)
                     for _ in range(CG))
        accs = lax.fori_loop(0, ntri, tri_body, acc0)
        for c in range(CG):
            out_v[gl, c * LANES:(c + 1) * LANES] = accs[c]

    pltpu.sync_copy(out_v, out_hbm.at[wid])


@jax.jit
def kernel(x, batch):
    starts = jnp.searchsorted(
        batch, jnp.arange(G + 1, dtype=jnp.int32), method="compare_all"
    ).astype(jnp.int32)
    starts = jnp.concatenate(
        [starts, jnp.full((STARTS_PAD - (G + 1),), N, jnp.int32)])

    fn = pl.kernel(
        _seg_max_body,
        out_type=jax.ShapeDtypeStruct((NW, SEG_PER_W, D), jnp.float32),
        mesh=plsc.VectorSubcoreMesh(core_axis_name="c", subcore_axis_name="s"),
        scratch_types=[
            pltpu.VMEM((STARTS_PAD,), jnp.int32),
            pltpu.VMEM((K, D), jnp.float32),
            pltpu.VMEM((K, D), jnp.float32),
            pltpu.VMEM((K, D), jnp.float32),
            pltpu.VMEM((SEG_PER_W, D), jnp.float32),
            pltpu.SemaphoreType.DMA,
            pltpu.SemaphoreType.DMA,
            pltpu.SemaphoreType.DMA,
        ],
    )
    return fn(x, starts).reshape(G, D)
